# Initial kernel scaffold; baseline (speedup 1.0000x reference)
import functools

import jax
import jax.numpy as jnp
from jax import lax
from jax.experimental import pallas as pl
from jax.experimental.pallas import tpu as pltpu
from jax.experimental.pallas import tpu_sc as plsc

B, L, V, D = 4096, 200, 100000, 128
NC, NS = 2, 16
NW = NC * NS
BPW = B // NW
LANES = 16
NG = D // LANES
C1 = 128
C2 = L - C1


def _bag(elements, sizes, table):
    mesh = plsc.VectorSubcoreMesh(core_axis_name="c", subcore_axis_name="s")

    @functools.partial(
        pl.kernel,
        out_type=jax.ShapeDtypeStruct((B, D), jnp.float32),
        mesh=mesh,
        scratch_types=[
            pltpu.VMEM((BPW, L), jnp.int32),
            pltpu.VMEM((L, D), jnp.float32),
            pltpu.VMEM((BPW, D), jnp.float32),
            pltpu.SMEM((BPW,), jnp.float32),
            pltpu.SemaphoreType.DMA,
        ],
    )
    def k(elements_hbm, sizes_hbm, table_hbm, out_hbm,
          idx_v, rows_v, out_v, sizes_s, sem):
        wid = lax.axis_index("s") * NC + lax.axis_index("c")
        base = wid * BPW
        pltpu.sync_copy(elements_hbm.at[pl.ds(base, BPW)], idx_v)
        pltpu.sync_copy(sizes_hbm.at[pl.ds(base, BPW)], sizes_s)

        @pl.loop(0, BPW)
        def _(i):
            cp1 = pltpu.async_copy(
                table_hbm.at[idx_v.at[i, pl.ds(0, C1)]],
                rows_v.at[pl.ds(0, C1)], sem)
            cp2 = pltpu.async_copy(
                table_hbm.at[idx_v.at[i, pl.ds(C1, C2)]],
                rows_v.at[pl.ds(C1, C2)], sem)
            cp1.wait()
            cp2.wait()

            inv = 1.0 / sizes_s[i]

            def body(j, acc):
                return tuple(acc[g] + rows_v[j, pl.ds(g * LANES, LANES)]
                             for g in range(NG))

            acc = lax.fori_loop(
                0, L, body,
                tuple(jnp.zeros((LANES,), jnp.float32) for _ in range(NG)))
            for g in range(NG):
                out_v[i, pl.ds(g * LANES, LANES)] = acc[g] * inv

        pltpu.sync_copy(out_v, out_hbm.at[pl.ds(base, BPW)])

    return k(elements, sizes, table)


def kernel(elements, sizes, table):
    return _bag(elements.astype(jnp.int32), sizes, table)


# speedup vs baseline: 7.7423x; 7.7423x over previous
import functools

import jax
import jax.numpy as jnp
from jax import lax
from jax.experimental import pallas as pl
from jax.experimental.pallas import tpu as pltpu
from jax.experimental.pallas import tpu_sc as plsc

B, L, V, D = 4096, 200, 100000, 128
NC, NS = 2, 16
NW = NC * NS
BPW = B // NW
LANES = 16
NG = D // LANES
C1 = 128
C2 = L - C1


def _bag(elements, sizes, table):
    mesh = plsc.VectorSubcoreMesh(core_axis_name="c", subcore_axis_name="s")

    @functools.partial(
        pl.kernel,
        out_type=jax.ShapeDtypeStruct((B, D), jnp.float32),
        mesh=mesh,
        scratch_types=[
            pltpu.VMEM((BPW, L), jnp.int32),
            pltpu.VMEM((L, D), jnp.float32),
            pltpu.VMEM((BPW, D), jnp.float32),
            pltpu.VMEM((BPW,), jnp.float32),
            pltpu.SemaphoreType.DMA,
        ],
    )
    def k(elements_hbm, sizes_hbm, table_hbm, out_hbm,
          idx_v, rows_v, out_v, sizes_s, sem):
        wid = lax.axis_index("s") * NC + lax.axis_index("c")
        base = wid * BPW
        pltpu.sync_copy(elements_hbm.at[pl.ds(base, BPW)], idx_v)
        pltpu.sync_copy(sizes_hbm.at[pl.ds(base, BPW)], sizes_s)

        @pl.loop(0, BPW // LANES)
        def _(gi):
            inv = 1.0 / sizes_s[pl.ds(gi * LANES, LANES)]
            for j in range(LANES):
                i = gi * LANES + j
                cp1 = pltpu.async_copy(
                    table_hbm.at[idx_v.at[i, pl.ds(0, C1)]],
                    rows_v.at[pl.ds(0, C1)], sem)
                cp2 = pltpu.async_copy(
                    table_hbm.at[idx_v.at[i, pl.ds(C1, C2)]],
                    rows_v.at[pl.ds(C1, C2)], sem)
                cp1.wait()
                cp2.wait()

                def body(jj, acc):
                    return tuple(acc[g] + rows_v[jj, pl.ds(g * LANES, LANES)]
                                 for g in range(NG))

                acc = lax.fori_loop(
                    0, L, body,
                    tuple(jnp.zeros((LANES,), jnp.float32)
                          for _ in range(NG)))
                for g in range(NG):
                    out_v[i, pl.ds(g * LANES, LANES)] = acc[g] * inv[j]

        pltpu.sync_copy(out_v, out_hbm.at[pl.ds(base, BPW)])

    return k(elements, sizes, table)


def kernel(elements, sizes, table):
    return _bag(elements.astype(jnp.int32), sizes, table)


# speedup vs baseline: 13.7112x; 1.7709x over previous
import functools

import jax
import jax.numpy as jnp
from jax import lax
from jax.experimental import pallas as pl
from jax.experimental.pallas import tpu as pltpu
from jax.experimental.pallas import tpu_sc as plsc

B, L, V, D = 4096, 200, 100000, 128
NC, NS = 2, 16
NW = NC * NS
BPW = B // NW
LANES = 16
NG = D // LANES
C1 = 128
C2 = L - C1


def _bag(elements, sizes, table):
    mesh = plsc.VectorSubcoreMesh(core_axis_name="c", subcore_axis_name="s")

    @functools.partial(
        pl.kernel,
        out_type=jax.ShapeDtypeStruct((B, D), jnp.float32),
        mesh=mesh,
        scratch_types=[
            pltpu.VMEM((BPW, L), jnp.int32),
            pltpu.VMEM((2, L, D), jnp.float32),
            pltpu.VMEM((BPW, D), jnp.float32),
            pltpu.VMEM((BPW,), jnp.float32),
            pltpu.SemaphoreType.DMA,
            pltpu.SemaphoreType.DMA,
        ],
    )
    def k(elements_hbm, sizes_hbm, table_hbm, out_hbm,
          idx_v, rows_v, out_v, sizes_s, sem0, sem1):
        wid = lax.axis_index("s") * NC + lax.axis_index("c")
        base = wid * BPW
        pltpu.sync_copy(elements_hbm.at[pl.ds(base, BPW)], idx_v)
        pltpu.sync_copy(sizes_hbm.at[pl.ds(base, BPW)], sizes_s)
        sems = (sem0, sem1)

        def gather_ops(i, buf):
            sem = sems[buf]
            return (
                pltpu.make_async_copy(
                    table_hbm.at[idx_v.at[i, pl.ds(0, C1)]],
                    rows_v.at[buf, pl.ds(0, C1)], sem),
                pltpu.make_async_copy(
                    table_hbm.at[idx_v.at[i, pl.ds(C1, C2)]],
                    rows_v.at[buf, pl.ds(C1, C2)], sem),
            )

        def issue(i, buf):
            for cp in gather_ops(i, buf):
                cp.start()

        def wait(buf):
            for cp in gather_ops(0, buf):
                cp.wait()

        issue(0, 0)

        @pl.loop(0, BPW // LANES)
        def _(gi):
            inv = 1.0 / sizes_s[pl.ds(gi * LANES, LANES)]
            for j in range(LANES):
                i = gi * LANES + j
                cur = j % 2
                nxt = 1 - cur

                @pl.when(i < BPW - 1)
                def _():
                    issue(i + 1, nxt)

                wait(cur)

                def body(jj, acc):
                    return tuple(
                        acc[g] + rows_v[cur, jj, pl.ds(g * LANES, LANES)]
                        for g in range(NG))

                acc = lax.fori_loop(
                    0, L, body,
                    tuple(jnp.zeros((LANES,), jnp.float32)
                          for _ in range(NG)),
                    unroll=4)
                for g in range(NG):
                    out_v[i, pl.ds(g * LANES, LANES)] = acc[g] * inv[j]

        pltpu.sync_copy(out_v, out_hbm.at[pl.ds(base, BPW)])

    return k(elements, sizes, table)


def kernel(elements, sizes, table):
    return _bag(elements.astype(jnp.int32), sizes, table)
